# baseline (speedup 1.0000x reference)
import functools

import jax
import jax.numpy as jnp
from jax import lax
from jax.experimental import pallas as pl
from jax.experimental.pallas import tpu as pltpu
from jax.experimental.pallas import tpu_sc as plsc

_C = 128
_NBUF = 5
_G = 3


@functools.lru_cache(maxsize=None)
def _gather_kernel(B, L, V, D, NW):
    mesh = plsc.VectorSubcoreMesh(core_axis_name="c", subcore_axis_name="s")

    @functools.partial(
        pl.kernel,
        out_type=jax.ShapeDtypeStruct((B * L, D), jnp.float32),
        mesh=mesh,
        scratch_types=[
            pltpu.VMEM((L, _C), jnp.int32),
            *[pltpu.VMEM((_C, D), jnp.float32) for _ in range(_NBUF)],
            *[pltpu.SemaphoreType.DMA for _ in range(2 * _NBUF)],
        ],
        compiler_params=pltpu.CompilerParams(use_tc_tiling_on_sc=True),
    )
    def k(table_hbm, idx_hbm, out_hbm, idx_v, *rest):
        bufs = rest[:_NBUF]
        gsem = rest[_NBUF:2 * _NBUF]
        wsem = rest[2 * _NBUF:3 * _NBUF]
        wid = lax.axis_index("s") * 2 + lax.axis_index("c")
        col = wid * _C

        pltpu.sync_copy(idx_hbm.at[:, pl.ds(col, _C)], idx_v)

        for s in range(_G):
            pltpu.async_copy(table_hbm.at[idx_v.at[s]], bufs[s], gsem[s])

        @pl.loop(0, L // _NBUF)
        def _(i):
            for s in range(_NBUF):
                ch = i * _NBUF + s
                pltpu.make_async_copy(
                    table_hbm.at[pl.ds(0, _C)], bufs[s], gsem[s]).wait()
                pltpu.async_copy(
                    bufs[s], out_hbm.at[pl.ds(ch * B + col, _C)], wsem[s])
                nxt = ch + _G
                sn = (s + _G) % _NBUF

                @pl.when(jnp.logical_and(nxt < L, nxt >= _NBUF))
                def _():
                    pltpu.make_async_copy(
                        bufs[sn], out_hbm.at[pl.ds(col, _C)], wsem[sn]).wait()

                @pl.when(nxt < L)
                def _():
                    pltpu.async_copy(
                        table_hbm.at[idx_v.at[nxt]], bufs[sn], gsem[sn])

        for s in range(_NBUF):
            pltpu.make_async_copy(
                bufs[s], out_hbm.at[pl.ds(col, _C)], wsem[s]).wait()

    return k


def kernel(input, table):
    B, L = input.shape
    V, D = table.shape
    NW = 32
    idx = input.T.astype(jnp.int32)
    out = _gather_kernel(B, L, V, D, NW)(table, idx)
    return out.reshape(L, B, D).transpose(1, 0, 2)


# speedup vs baseline: 1.0016x; 1.0016x over previous
import functools

import jax
import jax.numpy as jnp
from jax import lax
from jax.experimental import pallas as pl
from jax.experimental.pallas import tpu as pltpu
from jax.experimental.pallas import tpu_sc as plsc

_C = 128
_NBUF = 5


@functools.lru_cache(maxsize=None)
def _gather_kernel(B, L, V, D, NW):
    mesh = plsc.VectorSubcoreMesh(core_axis_name="c", subcore_axis_name="s")

    @functools.partial(
        pl.kernel,
        out_type=jax.ShapeDtypeStruct((B * L, D), jnp.float32),
        mesh=mesh,
        scratch_types=[
            pltpu.VMEM((L, _C), jnp.int32),
            *[pltpu.VMEM((_C, D), jnp.float32) for _ in range(_NBUF)],
            *[pltpu.SemaphoreType.DMA for _ in range(2 * _NBUF)],
        ],
        compiler_params=pltpu.CompilerParams(use_tc_tiling_on_sc=True),
    )
    def k(table_hbm, idx_hbm, out_hbm, idx_v, *rest):
        bufs = rest[:_NBUF]
        gsem = rest[_NBUF:2 * _NBUF]
        wsem = rest[2 * _NBUF:3 * _NBUF]
        wid = lax.axis_index("s") * 2 + lax.axis_index("c")
        col = wid * _C

        pltpu.sync_copy(idx_hbm.at[:, pl.ds(col, _C)], idx_v)

        for s in range(_NBUF):
            pltpu.async_copy(table_hbm.at[idx_v.at[s]], bufs[s], gsem[s])

        @pl.loop(0, L // _NBUF)
        def _(i):
            for s in range(_NBUF):
                ch = i * _NBUF + s
                pltpu.make_async_copy(
                    table_hbm.at[pl.ds(0, _C)], bufs[s], gsem[s]).wait()
                pltpu.async_copy(
                    bufs[s], out_hbm.at[pl.ds(ch * B + col, _C)], wsem[s])
                nxt = ch + _NBUF

                @pl.when(nxt < L)
                def _():
                    pltpu.make_async_copy(
                        bufs[s], out_hbm.at[pl.ds(col, _C)], wsem[s]).wait()
                    pltpu.async_copy(
                        table_hbm.at[idx_v.at[nxt]], bufs[s], gsem[s])

        for s in range(_NBUF):
            pltpu.make_async_copy(
                bufs[s], out_hbm.at[pl.ds(col, _C)], wsem[s]).wait()

    return k


def kernel(input, table):
    B, L = input.shape
    V, D = table.shape
    NW = 32
    idx = input.T.astype(jnp.int32)
    out = _gather_kernel(B, L, V, D, NW)(table, idx)
    return out.reshape(L, B, D).transpose(1, 0, 2)
